# baseline (speedup 1.0000x reference)
import functools

import jax
import jax.numpy as jnp
from jax.experimental import pallas as pl

_IN_CH = [512, 1024, 512, 256, 256, 256]
_ANCH = [4, 6, 6, 6, 4, 4]
_SP = [38, 19, 10, 5, 3, 1]
_NC = 21


def _ceil(x, m):
    return (x + m - 1) // m * m


def _conv_body(x_ref, w_ref, b_ref, o_ref, *, n_rows, wp):
    acc = jnp.zeros(o_ref.shape[1:], jnp.float32)
    for k in range(9):
        off = (k // 3) * wp + (k % 3)
        acc = acc + jax.lax.dot_general(
            x_ref[0, off:off + n_rows, :], w_ref[k],
            (((1,), (0,)), ((), ())), preferred_element_type=jnp.float32)
    o_ref[0] = acc + b_ref[...]


def _level(x, reg_w, reg_b, cls_w, cls_b):
    b, c, h, w = x.shape
    a = reg_w.shape[0] // 4
    cout = 25 * a
    coutp = _ceil(cout, 128)
    wp = w + 2
    n_rows = _ceil(h * wp, 8)
    x_rows = _ceil(n_rows + 2 * wp + 2, 8)

    xt = jnp.transpose(x, (0, 2, 3, 1))
    xp = jnp.pad(xt, ((0, 0), (1, 1), (1, wp - w - 1), (0, 0)))
    xf = xp.reshape(b, (h + 2) * wp, c)
    xf = jnp.pad(xf, ((0, 0), (0, x_rows - (h + 2) * wp), (0, 0)))

    wcat = jnp.concatenate([reg_w, cls_w], axis=0)
    wk = jnp.transpose(wcat, (2, 3, 1, 0)).reshape(9, c, cout)
    wk = jnp.pad(wk, ((0, 0), (0, 0), (0, coutp - cout)))
    bias = jnp.concatenate([reg_b, cls_b])[None, :]
    bias = jnp.pad(bias, ((0, 0), (0, coutp - cout)))

    out = pl.pallas_call(
        functools.partial(_conv_body, n_rows=n_rows, wp=wp),
        grid=(b,),
        in_specs=[
            pl.BlockSpec((1, x_rows, c), lambda i: (i, 0, 0)),
            pl.BlockSpec((9, c, coutp), lambda i: (0, 0, 0)),
            pl.BlockSpec((1, coutp), lambda i: (0, 0)),
        ],
        out_specs=pl.BlockSpec((1, n_rows, coutp), lambda i: (i, 0, 0)),
        out_shape=jax.ShapeDtypeStruct((b, n_rows, coutp), jnp.float32),
    )(xf, wk, bias)

    y = out[:, :h * wp, :].reshape(b, h, wp, coutp)[:, :, :w, :]
    reg = y[..., :4 * a].reshape(b, h * w * a, 4)
    cls = y[..., 4 * a:cout].reshape(b, h * w * a, _NC)
    return reg, cls


def kernel(x0, x1, x2, x3, x4, x5, reg_w0, reg_b0, cls_w0, cls_b0, reg_w1, reg_b1, cls_w1, cls_b1, reg_w2, reg_b2, cls_w2, cls_b2, reg_w3, reg_b3, cls_w3, cls_b3, reg_w4, reg_b4, cls_w4, cls_b4, reg_w5, reg_b5, cls_w5, cls_b5):
    xs = [x0, x1, x2, x3, x4, x5]
    rws = [reg_w0, reg_w1, reg_w2, reg_w3, reg_w4, reg_w5]
    rbs = [reg_b0, reg_b1, reg_b2, reg_b3, reg_b4, reg_b5]
    cws = [cls_w0, cls_w1, cls_w2, cls_w3, cls_w4, cls_w5]
    cbs = [cls_b0, cls_b1, cls_b2, cls_b3, cls_b4, cls_b5]
    regs, clss = [], []
    for i in range(6):
        r, cl = _level(xs[i], rws[i], rbs[i], cws[i], cbs[i])
        regs.append(r)
        clss.append(cl)
    bbox = jnp.concatenate(regs, axis=1)
    cls = jnp.concatenate(clss, axis=1)
    return (bbox, cls)


# speedup vs baseline: 1.0794x; 1.0794x over previous
import functools

import jax
import jax.numpy as jnp
from jax.experimental import pallas as pl

_IN_CH = [512, 1024, 512, 256, 256, 256]
_ANCH = [4, 6, 6, 6, 4, 4]
_SP = [38, 19, 10, 5, 3, 1]
_NC = 21


def _ceil(x, m):
    return (x + m - 1) // m * m


def _conv_body(x_ref, w_ref, b_ref, o_ref, *, n_rows, wp):
    acc = jnp.zeros(o_ref.shape[1:], jnp.float32)
    for k in range(9):
        off = (k // 3) * wp + (k % 3)
        acc = acc + jax.lax.dot_general(
            x_ref[0, off:off + n_rows, :], w_ref[k],
            (((1,), (0,)), ((), ())), preferred_element_type=jnp.float32)
    o_ref[0] = acc + b_ref[...]


def _level(x, reg_w, reg_b, cls_w, cls_b):
    b, c, h, w = x.shape
    a = reg_w.shape[0] // 4
    cout = 25 * a
    coutp = _ceil(cout, 128)
    wp = w + 2
    n_rows = _ceil(h * wp, 8)
    x_rows = _ceil(n_rows + 2 * wp + 2, 8)

    xt = jnp.transpose(x.astype(jnp.bfloat16), (0, 2, 3, 1))
    xp = jnp.pad(xt, ((0, 0), (1, 1), (1, wp - w - 1), (0, 0)))
    xf = xp.reshape(b, (h + 2) * wp, c)
    xf = jnp.pad(xf, ((0, 0), (0, x_rows - (h + 2) * wp), (0, 0)))

    wcat = jnp.concatenate([reg_w, cls_w], axis=0)
    wk = jnp.transpose(wcat, (2, 3, 1, 0)).reshape(9, c, cout)
    wk = jnp.pad(wk, ((0, 0), (0, 0), (0, coutp - cout))).astype(jnp.bfloat16)
    bias = jnp.concatenate([reg_b, cls_b])[None, :]
    bias = jnp.pad(bias, ((0, 0), (0, coutp - cout)))

    out = pl.pallas_call(
        functools.partial(_conv_body, n_rows=n_rows, wp=wp),
        grid=(b,),
        in_specs=[
            pl.BlockSpec((1, x_rows, c), lambda i: (i, 0, 0)),
            pl.BlockSpec((9, c, coutp), lambda i: (0, 0, 0)),
            pl.BlockSpec((1, coutp), lambda i: (0, 0)),
        ],
        out_specs=pl.BlockSpec((1, n_rows, coutp), lambda i: (i, 0, 0)),
        out_shape=jax.ShapeDtypeStruct((b, n_rows, coutp), jnp.float32),
    )(xf, wk, bias)

    y = out[:, :h * wp, :].reshape(b, h, wp, coutp)[:, :, :w, :]
    reg = y[..., :4 * a].reshape(b, h * w * a, 4)
    cls = y[..., 4 * a:cout].reshape(b, h * w * a, _NC)
    return reg, cls


def kernel(x0, x1, x2, x3, x4, x5, reg_w0, reg_b0, cls_w0, cls_b0, reg_w1, reg_b1, cls_w1, cls_b1, reg_w2, reg_b2, cls_w2, cls_b2, reg_w3, reg_b3, cls_w3, cls_b3, reg_w4, reg_b4, cls_w4, cls_b4, reg_w5, reg_b5, cls_w5, cls_b5):
    xs = [x0, x1, x2, x3, x4, x5]
    rws = [reg_w0, reg_w1, reg_w2, reg_w3, reg_w4, reg_w5]
    rbs = [reg_b0, reg_b1, reg_b2, reg_b3, reg_b4, reg_b5]
    cws = [cls_w0, cls_w1, cls_w2, cls_w3, cls_w4, cls_w5]
    cbs = [cls_b0, cls_b1, cls_b2, cls_b3, cls_b4, cls_b5]
    regs, clss = [], []
    for i in range(6):
        r, cl = _level(xs[i], rws[i], rbs[i], cws[i], cbs[i])
        regs.append(r)
        clss.append(cl)
    bbox = jnp.concatenate(regs, axis=1)
    cls = jnp.concatenate(clss, axis=1)
    return (bbox, cls)


# speedup vs baseline: 1.2509x; 1.1588x over previous
import functools

import jax
import jax.numpy as jnp
from jax.experimental import pallas as pl

_NC = 21


def _ceil(x, m):
    return (x + m - 1) // m * m


def _conv_body(x_ref, w_ref, b_ref, o_ref, *, h, w, coutp):
    hw = h * w
    c = x_ref.shape[1]
    f32 = jnp.float32
    xv = x_ref[0]
    dhs = (-1, 0, 1) if h > 1 else (0,)
    dws = (-1, 0, 1) if w > 1 else (0,)
    xs = {0: xv}
    if h > 1:
        z = jnp.zeros((c, w), f32)
        xs[-1] = jnp.concatenate([z, xv], axis=1)[:, :hw]
        xs[1] = jnp.concatenate([xv, z], axis=1)[:, w:]

    def mm(k, rhs):
        return jax.lax.dot_general(w_ref[k], rhs, (((1,), (0,)), ((), ())),
                                   preferred_element_type=f32)

    out = None
    for dw in dws:
        acc = None
        for dh in dhs:
            t = mm((dh + 1) * 3 + (dw + 1), xs[dh])
            acc = t if acc is None else acc + t
        if dw != 0:
            lane = jax.lax.broadcasted_iota(jnp.int32, (1, hw), 1)
            zc = jnp.zeros((coutp, 1), f32)
            if dw == -1:
                acc = jnp.where(lane % w != 0,
                                jnp.concatenate([zc, acc], axis=1)[:, :hw], 0.0)
            else:
                acc = jnp.where(lane % w != w - 1,
                                jnp.concatenate([acc, zc], axis=1)[:, 1:], 0.0)
        out = acc if out is None else out + acc
    o_ref[0] = out + b_ref[...]


def _mm_body(x_ref, w_ref, b_ref, o_ref):
    o_ref[...] = jax.lax.dot_general(
        w_ref[...], x_ref[...], (((1,), (0,)), ((), ())),
        preferred_element_type=jnp.float32) + b_ref[...]


def _level(x, reg_w, reg_b, cls_w, cls_b):
    b, c, h, w = x.shape
    hw = h * w
    a = reg_w.shape[0] // 4
    cout = 25 * a
    coutp = _ceil(cout, 8)

    wcat = jnp.concatenate([reg_w, cls_w], axis=0)
    bias = jnp.concatenate([reg_b, cls_b])
    bias = jnp.pad(bias, (0, coutp - cout))[:, None]

    if hw == 1:
        wc = jnp.pad(wcat[:, :, 1, 1], ((0, coutp - cout), (0, 0)))
        xt = jnp.transpose(x.reshape(b, c), (1, 0))
        out = pl.pallas_call(
            _mm_body,
            out_shape=jax.ShapeDtypeStruct((coutp, b), jnp.float32),
        )(xt, wc, bias)
        y = jnp.transpose(out, (1, 0))
        reg = y[:, :4 * a].reshape(b, a, 4)
        cls = y[:, 4 * a:cout].reshape(b, a, _NC)
        return reg, cls

    wk = jnp.transpose(wcat, (2, 3, 0, 1)).reshape(9, cout, c)
    wk = jnp.pad(wk, ((0, 0), (0, coutp - cout), (0, 0)))

    xf = x.reshape(b, c, hw)
    out = pl.pallas_call(
        functools.partial(_conv_body, h=h, w=w, coutp=coutp),
        grid=(b,),
        in_specs=[
            pl.BlockSpec((1, c, hw), lambda i: (i, 0, 0)),
            pl.BlockSpec((9, coutp, c), lambda i: (0, 0, 0)),
            pl.BlockSpec((coutp, 1), lambda i: (0, 0)),
        ],
        out_specs=pl.BlockSpec((1, coutp, hw), lambda i: (i, 0, 0)),
        out_shape=jax.ShapeDtypeStruct((b, coutp, hw), jnp.float32),
    )(xf, wk, bias)

    y = jnp.transpose(out, (0, 2, 1))
    reg = y[..., :4 * a].reshape(b, hw * a, 4)
    cls = y[..., 4 * a:cout].reshape(b, hw * a, _NC)
    return reg, cls


def kernel(x0, x1, x2, x3, x4, x5, reg_w0, reg_b0, cls_w0, cls_b0, reg_w1, reg_b1, cls_w1, cls_b1, reg_w2, reg_b2, cls_w2, cls_b2, reg_w3, reg_b3, cls_w3, cls_b3, reg_w4, reg_b4, cls_w4, cls_b4, reg_w5, reg_b5, cls_w5, cls_b5):
    xs = [x0, x1, x2, x3, x4, x5]
    rws = [reg_w0, reg_w1, reg_w2, reg_w3, reg_w4, reg_w5]
    rbs = [reg_b0, reg_b1, reg_b2, reg_b3, reg_b4, reg_b5]
    cws = [cls_w0, cls_w1, cls_w2, cls_w3, cls_w4, cls_w5]
    cbs = [cls_b0, cls_b1, cls_b2, cls_b3, cls_b4, cls_b5]
    regs, clss = [], []
    for i in range(6):
        r, cl = _level(xs[i], rws[i], rbs[i], cws[i], cbs[i])
        regs.append(r)
        clss.append(cl)
    bbox = jnp.concatenate(regs, axis=1)
    cls = jnp.concatenate(clss, axis=1)
    return (bbox, cls)
